# baseline (device time: 83381 ns/iter reference)
import jax
import jax.numpy as jnp
import numpy as np
from jax import lax
from jax.experimental import pallas as pl
from jax.experimental.pallas import tpu as pltpu

N = 32
B, S, C = 4, 1024, 512
OUT_N = 512
ROWS = B * S
TAPS = 4

NA = 8
NB = 4
AC = ROWS // NA
H = AC // 2
NQ = 2
QH = H // NQ
BC = H // NB


def _build_tables():
    fb = ([p // NB for p in range(N)],
          [((p // NB + 1) % NA) * NB + p % NB for p in range(N)],
          [((p // NB - 1) % NA) * NB + p % NB for p in range(N)],
          [p % NB for p in range(N)],
          [(p // NB) * NB + (p + 1) % NB for p in range(N)],
          [(p // NB) * NB + (p - 1) % NB for p in range(N)])
    try:
        import distributed_mesh_v7x as dm
        mesh = dm.get_mesh("i", world_size=N)
        devs = list(mesh.devices.flat)
        coords = [tuple(d.coords) for d in devs]
        if len(set(coords)) != N or any(len(c) != 3 for c in coords):
            return fb
        axes = [sorted({c[i] for c in coords}) for i in range(3)]
        sizes = [len(a) for a in axes]
        if sorted(sizes) != [2, 4, 4]:
            return fb
        a2 = sizes.index(2)
        a4 = [i for i in range(3) if i != a2]
        us = axes[a4[0]]
        _v = axes[a4[1]]
        vs = [_v[0], _v[1], _v[3], _v[2]]
        lo, hi = axes[a2]
        cyc = [(lo, u) for u in us] + [(hi, u) for u in reversed(us)]
        posA_of = {xu: i for i, xu in enumerate(cyc)}
        log_of = {c: p for p, c in enumerate(coords)}

        def at(c, i2, iu, iv):
            t = [0, 0, 0]
            t[a2], t[a4[0]], t[a4[1]] = i2, iu, iv
            return tuple(t)

        posA = [0] * N
        rtA = [0] * N
        ltA = [0] * N
        posB = [0] * N
        rtB = [0] * N
        ltB = [0] * N
        for p, c in enumerate(coords):
            i2, iu, iv = c[a2], c[a4[0]], c[a4[1]]
            pa = posA_of[(i2, iu)]
            pb = vs.index(iv)
            posA[p] = pa
            posB[p] = pb
            nxt = cyc[(pa + 1) % NA]
            prv = cyc[(pa - 1) % NA]
            rtA[p] = log_of[at(c, nxt[0], nxt[1], iv)]
            ltA[p] = log_of[at(c, prv[0], prv[1], iv)]
            rtB[p] = log_of[at(c, i2, iu, vs[(pb + 1) % NB])]
            ltB[p] = log_of[at(c, i2, iu, vs[(pb - 1) % NB])]
        return posA, rtA, ltA, posB, rtB, ltB
    except Exception:
        return fb


_POSA, _RTA, _LTA, _POSB, _RTB, _LTB = _build_tables()


def _body(scal_ref, x_ref, k_ref, wp_ref, out_ref,
          pad_ref, part_ref, obf_ref, bufAR, bufAL, bufBR, bufBL,
          sbAR, sbAL, sbBR, sbBL,
          rsA_sR, rsA_rR, rsA_sL, rsA_rL,
          rsB_sR, rsB_rR, rsB_sL, rsB_rL,
          agB_sR, agB_rR, agB_sL, agB_rL,
          agA_sR, agA_rR, agA_sL, agA_rL,
          stage_ref, cp_sems, x_sems):
    my = lax.axis_index("i")
    posA = scal_ref[0, my]
    rtA = scal_ref[1, my]
    ltA = scal_ref[2, my]
    posB = scal_ref[3, my]
    rtB = scal_ref[4, my]
    ltB = scal_ref[5, my]

    barrier_sem = pltpu.get_barrier_semaphore()
    for nbr in (rtA, ltA, rtB, ltB):
        pl.semaphore_signal(barrier_sem, inc=1, device_id=(nbr,),
                            device_id_type=pl.DeviceIdType.MESH)

    kv = k_ref[:, :]
    wpv = wp_ref[:, :]
    xcps = []
    for b in range(B):
        pad_ref[b, 0:8, :] = jnp.zeros((8, C), jnp.float32)
        cp = pltpu.make_async_copy(
            x_ref.at[b], pad_ref.at[b, pl.ds(8, S), :], x_sems.at[b])
        cp.start()
        xcps.append(cp)
    for cp in xcps:
        cp.wait()

    def compute_half(c, half):
        b = c // 2
        rl = (c % 2) * AC + half * H
        w = pad_ref[b, pl.ds(rl, H + 16), :]
        acc = w[5:5 + H, :] * kv[0:1, :]
        for t in range(1, TAPS):
            acc = acc + w[5 + t:5 + t + H, :] * kv[t:t + 1, :]
        a = acc * (1.0 / (1.0 + jnp.exp(-acc)))
        part_ref[pl.ds(c * AC + half * H, H), :] = jnp.dot(
            a, wpv, preferred_element_type=jnp.float32)

    compute_half(posA % NA, 0)
    compute_half(posA % NA, 1)

    pl.semaphore_wait(barrier_sem, 4)

    def copy(src, dst, ssem, rsem, dev):
        r = pltpu.make_async_remote_copy(
            src_ref=src, dst_ref=dst, send_sem=ssem, recv_sem=rsem,
            device_id=dev, device_id_type=pl.DeviceIdType.LOGICAL)
        r.start()
        return r

    NSLOT = 4
    cps = []

    def emit(row, n):
        slot = len(cps) % NSLOT
        if len(cps) >= NSLOT:
            cps[len(cps) - NSLOT].wait()
        stage_ref[slot, 0:n, :] = obf_ref[pl.ds(row, n), :].astype(
            jnp.float32)
        cp = pltpu.make_async_copy(
            stage_ref.at[slot, pl.ds(0, n), :],
            out_ref.at[row // S, pl.ds(row % S, n), :],
            cp_sems.at[slot])
        cp.start()
        cps.append(cp)

    sbAR[0] = part_ref[pl.ds((posA % NA) * AC, H), :].astype(jnp.bfloat16)
    sbAL[0] = part_ref[pl.ds((posA % NA) * AC + H, H), :].astype(jnp.bfloat16)

    def rsA_send(d, q, s):
        sb, buf, ss, rs, dev = (
            (sbAR, bufAR, rsA_sR, rsA_rR, rtA) if d == 0 else
            (sbAL, bufAL, rsA_sL, rsA_rL, ltA))
        return copy(sb.at[s, pl.ds(q * QH, QH), :],
                    buf.at[s, pl.ds(q * QH, QH), :],
                    ss.at[NQ * s + q], rs.at[NQ * s + q], dev)

    rbase = ((posA + 1) % NA) * AC
    lbase = ((posA - 1) % NA) * AC + H
    rsB = [[None, None] for _ in range(NB - 1)]

    dR = [[rsA_send(0, q, 0) for q in range(NQ)]]
    dL = [[rsA_send(1, q, 0) for q in range(NQ)]]
    compute_half((posA - 1) % NA, 0)
    compute_half((posA + 1) % NA, 1)
    for s in range(NA - 1):
        crR = ((posA - s - 1) % NA) * AC
        crL = ((posA + s + 1) % NA) * AC + H
        if s + 1 < NA - 1:
            dR.append([None] * NQ)
            dL.append([None] * NQ)
        for q in range(NQ):
            dR[s][q].wait_recv()
            rows = pl.ds(crR + q * QH, QH)
            v = part_ref[rows, :] + bufAR[s, q * QH:(q + 1) * QH, :].astype(
                jnp.float32)
            if s < NA - 2:
                sbAR[s + 1, q * QH:(q + 1) * QH, :] = v.astype(jnp.bfloat16)
                dR[s + 1][q] = rsA_send(0, q, s + 1)
            else:
                part_ref[rows, :] = v
        if s == NA - 2:
            sbBR[0] = part_ref[pl.ds(rbase + (posB % NB) * BC, BC),
                               :].astype(jnp.bfloat16)
            rsB[0][0] = copy(sbBR.at[0], bufBR.at[0],
                             rsB_sR.at[0], rsB_rR.at[0], rtB)
        for q in range(NQ):
            dL[s][q].wait_recv()
            rows = pl.ds(crL + q * QH, QH)
            v = part_ref[rows, :] + bufAL[s, q * QH:(q + 1) * QH, :].astype(
                jnp.float32)
            if s < NA - 2:
                sbAL[s + 1, q * QH:(q + 1) * QH, :] = v.astype(jnp.bfloat16)
                dL[s + 1][q] = rsA_send(1, q, s + 1)
            else:
                part_ref[rows, :] = v
        if s == NA - 2:
            sbBL[0] = part_ref[pl.ds(lbase + (posB % NB) * BC, BC),
                               :].astype(jnp.bfloat16)
            rsB[0][1] = copy(sbBL.at[0], bufBL.at[0],
                             rsB_sL.at[0], rsB_rL.at[0], ltB)
        if s < NA - 2:
            compute_half((posA - s - 2) % NA, 0)
            compute_half((posA + s + 2) % NA, 1)

    for s in range(NB - 1):
        aR = rbase + ((posB - s - 1) % NB) * BC
        aL = lbase + ((posB + s + 1) % NB) * BC
        rR, rL = rsB[s]
        rR.wait_recv()
        vR = part_ref[pl.ds(aR, BC), :] + bufBR[s].astype(jnp.float32)
        if s < NB - 2:
            sbBR[s + 1] = vR.astype(jnp.bfloat16)
            rsB[s + 1][0] = copy(sbBR.at[s + 1], bufBR.at[s + 1],
                                 rsB_sR.at[s + 1], rsB_rR.at[s + 1], rtB)
        else:
            part_ref[pl.ds(aR, BC), :] = vR
        rL.wait_recv()
        vL = part_ref[pl.ds(aL, BC), :] + bufBL[s].astype(jnp.float32)
        if s < NB - 2:
            sbBL[s + 1] = vL.astype(jnp.bfloat16)
            rsB[s + 1][1] = copy(sbBL.at[s + 1], bufBL.at[s + 1],
                                 rsB_sL.at[s + 1], rsB_rL.at[s + 1], ltB)
        else:
            part_ref[pl.ds(aL, BC), :] = vL

    ownR = rbase + ((posB + 1) % NB) * BC
    ownL = lbase + ((posB - 1) % NB) * BC
    obf_ref[pl.ds(ownR, BC), :] = part_ref[pl.ds(ownR, BC), :].astype(
        jnp.bfloat16)
    obf_ref[pl.ds(ownL, BC), :] = part_ref[pl.ds(ownL, BC), :].astype(
        jnp.bfloat16)

    def agA_send(d, k, s):
        if d == 0:
            blk = (posB + 1 - k) % NB
            rows = pl.ds(((posA + 1 - s) % NA) * AC + blk * BC, BC)
            ss, rs, dev = agA_sR, agA_rR, rtA
        else:
            blk = (posB - 1 + k) % NB
            rows = pl.ds(((posA - 1 + s) % NA) * AC + H + blk * BC, BC)
            ss, rs, dev = agA_sL, agA_rL, ltA
        return copy(obf_ref.at[rows, :], obf_ref.at[rows, :],
                    ss.at[NB * s + k], rs.at[NB * s + k], dev)

    gR = [[None] * NB for _ in range(NA - 1)]
    gL = [[None] * NB for _ in range(NA - 1)]
    gR[0][0] = agA_send(0, 0, 0)
    gL[0][0] = agA_send(1, 0, 0)
    agB = []

    for s in range(NB - 1):
        sR = rbase + ((posB + 1 - s) % NB) * BC
        sL = lbase + ((posB - 1 + s) % NB) * BC
        rR = copy(obf_ref.at[pl.ds(sR, BC), :], obf_ref.at[pl.ds(sR, BC), :],
                  agB_sR.at[s], agB_rR.at[s], rtB)
        rL = copy(obf_ref.at[pl.ds(sL, BC), :], obf_ref.at[pl.ds(sL, BC), :],
                  agB_sL.at[s], agB_rL.at[s], ltB)
        rR.wait_recv()
        gR[0][s + 1] = agA_send(0, s + 1, 0)
        rL.wait_recv()
        gL[0][s + 1] = agA_send(1, s + 1, 0)
        agB.append((rR, rL))

    emit(rbase, H)
    emit(lbase, H)

    for s in range(NA - 1):
        for k in range(NB):
            gR[s][k].wait_recv()
            if s + 1 < NA - 1:
                gR[s + 1][k] = agA_send(0, k, s + 1)
        for k in range(NB):
            gL[s][k].wait_recv()
            if s + 1 < NA - 1:
                gL[s + 1][k] = agA_send(1, k, s + 1)
        emit(((posA - s) % NA) * AC, H)
        emit(((posA + s) % NA) * AC + H, H)

    for s in range(NA - 1):
        for q in range(NQ):
            dR[s][q].wait_send()
            dL[s][q].wait_send()
    for s in range(NB - 1):
        rsB[s][0].wait_send()
        rsB[s][1].wait_send()
        agB[s][0].wait_send()
        agB[s][1].wait_send()
    for s in range(NA - 1):
        for k in range(NB):
            gR[s][k].wait_send()
            gL[s][k].wait_send()
    for cp in cps[-NSLOT:]:
        cp.wait()


_TABLES = np.asarray([_POSA, _RTA, _LTA, _POSB, _RTB, _LTB], dtype=np.int32)


def kernel(x, k, Wp):
    scalars = jnp.asarray(_TABLES)

    sem7 = pltpu.SemaphoreType.DMA((NQ * (NA - 1),))
    sem3 = pltpu.SemaphoreType.DMA((NB - 1,))
    sem28 = pltpu.SemaphoreType.DMA((NB * (NA - 1),))
    out = pl.pallas_call(
        _body,
        out_shape=jax.ShapeDtypeStruct((B, S, OUT_N), jnp.float32),
        in_specs=[
            pl.BlockSpec(memory_space=pltpu.SMEM),
            pl.BlockSpec(memory_space=pltpu.MemorySpace.HBM),
            pl.BlockSpec(memory_space=pltpu.VMEM),
            pl.BlockSpec(memory_space=pltpu.VMEM),
        ],
        out_specs=pl.BlockSpec(memory_space=pltpu.MemorySpace.HBM),
        scratch_shapes=[
            pltpu.VMEM((B, S + 16, C), jnp.float32),
            pltpu.VMEM((ROWS, OUT_N), jnp.float32),
            pltpu.VMEM((ROWS, OUT_N), jnp.bfloat16),
            pltpu.VMEM((NA - 1, H, OUT_N), jnp.bfloat16),
            pltpu.VMEM((NA - 1, H, OUT_N), jnp.bfloat16),
            pltpu.VMEM((NB - 1, BC, OUT_N), jnp.bfloat16),
            pltpu.VMEM((NB - 1, BC, OUT_N), jnp.bfloat16),
            pltpu.VMEM((NA - 1, H, OUT_N), jnp.bfloat16),
            pltpu.VMEM((NA - 1, H, OUT_N), jnp.bfloat16),
            pltpu.VMEM((NB - 1, BC, OUT_N), jnp.bfloat16),
            pltpu.VMEM((NB - 1, BC, OUT_N), jnp.bfloat16),
            sem7, sem7, sem7, sem7,
            sem3, sem3, sem3, sem3,
            sem3, sem3, sem3, sem3,
            sem28, sem28, sem28, sem28,
            pltpu.VMEM((4, H, OUT_N), jnp.float32),
            pltpu.SemaphoreType.DMA((4,)),
            pltpu.SemaphoreType.DMA((B,)),
        ],
        compiler_params=pltpu.CompilerParams(collective_id=0),
    )(scalars, x, k, Wp)
    return out


# device time: 83035 ns/iter; 1.0042x vs baseline; 1.0042x over previous
import jax
import jax.numpy as jnp
import numpy as np
from jax import lax
from jax.experimental import pallas as pl
from jax.experimental.pallas import tpu as pltpu

N = 32
B, S, C = 4, 1024, 512
OUT_N = 512
ROWS = B * S
TAPS = 4

NA = 8
NB = 4
AC = ROWS // NA
H = AC // 2
NQ = 2
QH = H // NQ
BC = H // NB


def _build_tables():
    fb = ([p // NB for p in range(N)],
          [((p // NB + 1) % NA) * NB + p % NB for p in range(N)],
          [((p // NB - 1) % NA) * NB + p % NB for p in range(N)],
          [p % NB for p in range(N)],
          [(p // NB) * NB + (p + 1) % NB for p in range(N)],
          [(p // NB) * NB + (p - 1) % NB for p in range(N)])
    try:
        import distributed_mesh_v7x as dm
        mesh = dm.get_mesh("i", world_size=N)
        devs = list(mesh.devices.flat)
        coords = [tuple(d.coords) for d in devs]
        if len(set(coords)) != N or any(len(c) != 3 for c in coords):
            return fb
        axes = [sorted({c[i] for c in coords}) for i in range(3)]
        sizes = [len(a) for a in axes]
        if sorted(sizes) != [2, 4, 4]:
            return fb
        a2 = sizes.index(2)
        a4 = [i for i in range(3) if i != a2]
        us = axes[a4[0]]
        _v = axes[a4[1]]
        vs = [_v[0], _v[1], _v[3], _v[2]]
        lo, hi = axes[a2]
        cyc = [(lo, u) for u in us] + [(hi, u) for u in reversed(us)]
        posA_of = {xu: i for i, xu in enumerate(cyc)}
        log_of = {c: p for p, c in enumerate(coords)}

        def at(c, i2, iu, iv):
            t = [0, 0, 0]
            t[a2], t[a4[0]], t[a4[1]] = i2, iu, iv
            return tuple(t)

        posA = [0] * N
        rtA = [0] * N
        ltA = [0] * N
        posB = [0] * N
        rtB = [0] * N
        ltB = [0] * N
        for p, c in enumerate(coords):
            i2, iu, iv = c[a2], c[a4[0]], c[a4[1]]
            pa = posA_of[(i2, iu)]
            pb = vs.index(iv)
            posA[p] = pa
            posB[p] = pb
            nxt = cyc[(pa + 1) % NA]
            prv = cyc[(pa - 1) % NA]
            rtA[p] = log_of[at(c, nxt[0], nxt[1], iv)]
            ltA[p] = log_of[at(c, prv[0], prv[1], iv)]
            rtB[p] = log_of[at(c, i2, iu, vs[(pb + 1) % NB])]
            ltB[p] = log_of[at(c, i2, iu, vs[(pb - 1) % NB])]
        return posA, rtA, ltA, posB, rtB, ltB
    except Exception:
        return fb


_POSA, _RTA, _LTA, _POSB, _RTB, _LTB = _build_tables()


def _body(scal_ref, x_ref, k_ref, wp_ref, out_ref,
          pad_ref, part_ref, obf_ref, bufAR, bufAL, bufBR, bufBL,
          sbAR, sbAL, sbBR, sbBL,
          rsA_sR, rsA_rR, rsA_sL, rsA_rL,
          rsB_sR, rsB_rR, rsB_sL, rsB_rL,
          agB_sR, agB_rR, agB_sL, agB_rL,
          agA_sR, agA_rR, agA_sL, agA_rL,
          stage_ref, cp_sems, x_sems):
    my = lax.axis_index("i")
    posA = scal_ref[0, my]
    rtA = scal_ref[1, my]
    ltA = scal_ref[2, my]
    posB = scal_ref[3, my]
    rtB = scal_ref[4, my]
    ltB = scal_ref[5, my]

    barrier_sem = pltpu.get_barrier_semaphore()
    for nbr in (rtA, ltA, rtB, ltB):
        pl.semaphore_signal(barrier_sem, inc=1, device_id=(nbr,),
                            device_id_type=pl.DeviceIdType.MESH)

    kv = k_ref[:, :]
    wpv = wp_ref[:, :]
    xcps = []
    for b in range(B):
        pad_ref[b, 0:8, :] = jnp.zeros((8, C), jnp.float32)
        cp = pltpu.make_async_copy(
            x_ref.at[b], pad_ref.at[b, pl.ds(8, S), :], x_sems.at[b])
        cp.start()
        xcps.append(cp)
    for cp in xcps:
        cp.wait()

    def compute_half(c, half, stage=None):
        b = c // 2
        rl = (c % 2) * AC + half * H
        w = pad_ref[b, pl.ds(rl, H + 16), :]
        acc = w[5:5 + H, :] * kv[0:1, :]
        for t in range(1, TAPS):
            acc = acc + w[5 + t:5 + t + H, :] * kv[t:t + 1, :]
        a = acc * (1.0 / (1.0 + jnp.exp(-acc)))
        v = jnp.dot(a, wpv, preferred_element_type=jnp.float32)
        part_ref[pl.ds(c * AC + half * H, H), :] = v
        if stage is not None:
            stage[0] = v.astype(jnp.bfloat16)

    def copy(src, dst, ssem, rsem, dev):
        r = pltpu.make_async_remote_copy(
            src_ref=src, dst_ref=dst, send_sem=ssem, recv_sem=rsem,
            device_id=dev, device_id_type=pl.DeviceIdType.LOGICAL)
        r.start()
        return r

    NSLOT = 4
    cps = []

    def emit(row, n):
        slot = len(cps) % NSLOT
        if len(cps) >= NSLOT:
            cps[len(cps) - NSLOT].wait()
        stage_ref[slot, 0:n, :] = obf_ref[pl.ds(row, n), :].astype(
            jnp.float32)
        cp = pltpu.make_async_copy(
            stage_ref.at[slot, pl.ds(0, n), :],
            out_ref.at[row // S, pl.ds(row % S, n), :],
            cp_sems.at[slot])
        cp.start()
        cps.append(cp)

    def rsA_send(d, q, s):
        sb, buf, ss, rs, dev = (
            (sbAR, bufAR, rsA_sR, rsA_rR, rtA) if d == 0 else
            (sbAL, bufAL, rsA_sL, rsA_rL, ltA))
        return copy(sb.at[s, pl.ds(q * QH, QH), :],
                    buf.at[s, pl.ds(q * QH, QH), :],
                    ss.at[NQ * s + q], rs.at[NQ * s + q], dev)

    rbase = ((posA + 1) % NA) * AC
    lbase = ((posA - 1) % NA) * AC + H
    rsB = [[None, None] for _ in range(NB - 1)]

    compute_half(posA % NA, 0, stage=sbAR)
    pl.semaphore_wait(barrier_sem, 4)
    dR = [[rsA_send(0, q, 0) for q in range(NQ)]]
    compute_half(posA % NA, 1, stage=sbAL)
    dL = [[rsA_send(1, q, 0) for q in range(NQ)]]
    compute_half((posA - 1) % NA, 0)
    compute_half((posA + 1) % NA, 1)
    for s in range(NA - 1):
        crR = ((posA - s - 1) % NA) * AC
        crL = ((posA + s + 1) % NA) * AC + H
        if s + 1 < NA - 1:
            dR.append([None] * NQ)
            dL.append([None] * NQ)
        for q in range(NQ):
            dR[s][q].wait_recv()
            rows = pl.ds(crR + q * QH, QH)
            v = part_ref[rows, :] + bufAR[s, q * QH:(q + 1) * QH, :].astype(
                jnp.float32)
            if s < NA - 2:
                sbAR[s + 1, q * QH:(q + 1) * QH, :] = v.astype(jnp.bfloat16)
                dR[s + 1][q] = rsA_send(0, q, s + 1)
            else:
                part_ref[rows, :] = v
        if s == NA - 2:
            sbBR[0] = part_ref[pl.ds(rbase + (posB % NB) * BC, BC),
                               :].astype(jnp.bfloat16)
            rsB[0][0] = copy(sbBR.at[0], bufBR.at[0],
                             rsB_sR.at[0], rsB_rR.at[0], rtB)
        for q in range(NQ):
            dL[s][q].wait_recv()
            rows = pl.ds(crL + q * QH, QH)
            v = part_ref[rows, :] + bufAL[s, q * QH:(q + 1) * QH, :].astype(
                jnp.float32)
            if s < NA - 2:
                sbAL[s + 1, q * QH:(q + 1) * QH, :] = v.astype(jnp.bfloat16)
                dL[s + 1][q] = rsA_send(1, q, s + 1)
            else:
                part_ref[rows, :] = v
        if s == NA - 2:
            sbBL[0] = part_ref[pl.ds(lbase + (posB % NB) * BC, BC),
                               :].astype(jnp.bfloat16)
            rsB[0][1] = copy(sbBL.at[0], bufBL.at[0],
                             rsB_sL.at[0], rsB_rL.at[0], ltB)
        if s < NA - 2:
            compute_half((posA - s - 2) % NA, 0)
            compute_half((posA + s + 2) % NA, 1)

    for s in range(NB - 1):
        aR = rbase + ((posB - s - 1) % NB) * BC
        aL = lbase + ((posB + s + 1) % NB) * BC
        rR, rL = rsB[s]
        rR.wait_recv()
        vR = part_ref[pl.ds(aR, BC), :] + bufBR[s].astype(jnp.float32)
        if s < NB - 2:
            sbBR[s + 1] = vR.astype(jnp.bfloat16)
            rsB[s + 1][0] = copy(sbBR.at[s + 1], bufBR.at[s + 1],
                                 rsB_sR.at[s + 1], rsB_rR.at[s + 1], rtB)
        else:
            part_ref[pl.ds(aR, BC), :] = vR
        rL.wait_recv()
        vL = part_ref[pl.ds(aL, BC), :] + bufBL[s].astype(jnp.float32)
        if s < NB - 2:
            sbBL[s + 1] = vL.astype(jnp.bfloat16)
            rsB[s + 1][1] = copy(sbBL.at[s + 1], bufBL.at[s + 1],
                                 rsB_sL.at[s + 1], rsB_rL.at[s + 1], ltB)
        else:
            part_ref[pl.ds(aL, BC), :] = vL

    ownR = rbase + ((posB + 1) % NB) * BC
    ownL = lbase + ((posB - 1) % NB) * BC
    obf_ref[pl.ds(ownR, BC), :] = part_ref[pl.ds(ownR, BC), :].astype(
        jnp.bfloat16)
    obf_ref[pl.ds(ownL, BC), :] = part_ref[pl.ds(ownL, BC), :].astype(
        jnp.bfloat16)

    def agA_send(d, k, s):
        if d == 0:
            blk = (posB + 1 - k) % NB
            rows = pl.ds(((posA + 1 - s) % NA) * AC + blk * BC, BC)
            ss, rs, dev = agA_sR, agA_rR, rtA
        else:
            blk = (posB - 1 + k) % NB
            rows = pl.ds(((posA - 1 + s) % NA) * AC + H + blk * BC, BC)
            ss, rs, dev = agA_sL, agA_rL, ltA
        return copy(obf_ref.at[rows, :], obf_ref.at[rows, :],
                    ss.at[NB * s + k], rs.at[NB * s + k], dev)

    gR = [[None] * NB for _ in range(NA - 1)]
    gL = [[None] * NB for _ in range(NA - 1)]
    gR[0][0] = agA_send(0, 0, 0)
    gL[0][0] = agA_send(1, 0, 0)
    agB = []

    for s in range(NB - 1):
        sR = rbase + ((posB + 1 - s) % NB) * BC
        sL = lbase + ((posB - 1 + s) % NB) * BC
        rR = copy(obf_ref.at[pl.ds(sR, BC), :], obf_ref.at[pl.ds(sR, BC), :],
                  agB_sR.at[s], agB_rR.at[s], rtB)
        rL = copy(obf_ref.at[pl.ds(sL, BC), :], obf_ref.at[pl.ds(sL, BC), :],
                  agB_sL.at[s], agB_rL.at[s], ltB)
        rR.wait_recv()
        gR[0][s + 1] = agA_send(0, s + 1, 0)
        rL.wait_recv()
        gL[0][s + 1] = agA_send(1, s + 1, 0)
        agB.append((rR, rL))

    emit(rbase, H)
    emit(lbase, H)

    for s in range(NA - 1):
        for k in range(NB):
            gR[s][k].wait_recv()
            if s + 1 < NA - 1:
                gR[s + 1][k] = agA_send(0, k, s + 1)
        for k in range(NB):
            gL[s][k].wait_recv()
            if s + 1 < NA - 1:
                gL[s + 1][k] = agA_send(1, k, s + 1)
        emit(((posA - s) % NA) * AC, H)
        emit(((posA + s) % NA) * AC + H, H)

    for s in range(NA - 1):
        for q in range(NQ):
            dR[s][q].wait_send()
            dL[s][q].wait_send()
    for s in range(NB - 1):
        rsB[s][0].wait_send()
        rsB[s][1].wait_send()
        agB[s][0].wait_send()
        agB[s][1].wait_send()
    for s in range(NA - 1):
        for k in range(NB):
            gR[s][k].wait_send()
            gL[s][k].wait_send()
    for cp in cps[-NSLOT:]:
        cp.wait()


_TABLES = np.asarray([_POSA, _RTA, _LTA, _POSB, _RTB, _LTB], dtype=np.int32)


def kernel(x, k, Wp):
    scalars = jnp.asarray(_TABLES)

    sem7 = pltpu.SemaphoreType.DMA((NQ * (NA - 1),))
    sem3 = pltpu.SemaphoreType.DMA((NB - 1,))
    sem28 = pltpu.SemaphoreType.DMA((NB * (NA - 1),))
    out = pl.pallas_call(
        _body,
        out_shape=jax.ShapeDtypeStruct((B, S, OUT_N), jnp.float32),
        in_specs=[
            pl.BlockSpec(memory_space=pltpu.SMEM),
            pl.BlockSpec(memory_space=pltpu.MemorySpace.HBM),
            pl.BlockSpec(memory_space=pltpu.VMEM),
            pl.BlockSpec(memory_space=pltpu.VMEM),
        ],
        out_specs=pl.BlockSpec(memory_space=pltpu.MemorySpace.HBM),
        scratch_shapes=[
            pltpu.VMEM((B, S + 16, C), jnp.float32),
            pltpu.VMEM((ROWS, OUT_N), jnp.float32),
            pltpu.VMEM((ROWS, OUT_N), jnp.bfloat16),
            pltpu.VMEM((NA - 1, H, OUT_N), jnp.bfloat16),
            pltpu.VMEM((NA - 1, H, OUT_N), jnp.bfloat16),
            pltpu.VMEM((NB - 1, BC, OUT_N), jnp.bfloat16),
            pltpu.VMEM((NB - 1, BC, OUT_N), jnp.bfloat16),
            pltpu.VMEM((NA - 1, H, OUT_N), jnp.bfloat16),
            pltpu.VMEM((NA - 1, H, OUT_N), jnp.bfloat16),
            pltpu.VMEM((NB - 1, BC, OUT_N), jnp.bfloat16),
            pltpu.VMEM((NB - 1, BC, OUT_N), jnp.bfloat16),
            sem7, sem7, sem7, sem7,
            sem3, sem3, sem3, sem3,
            sem3, sem3, sem3, sem3,
            sem28, sem28, sem28, sem28,
            pltpu.VMEM((4, H, OUT_N), jnp.float32),
            pltpu.SemaphoreType.DMA((4,)),
            pltpu.SemaphoreType.DMA((B,)),
        ],
        compiler_params=pltpu.CompilerParams(collective_id=0),
    )(scalars, x, k, Wp)
    return out
